# Initial kernel scaffold; baseline (speedup 1.0000x reference)
import jax
import jax.numpy as jnp
from jax.experimental import pallas as pl


def kernel(x_licitacion, x_empresa, ei_participa, ei_similar, Wl_participa, bl_participa, Wr_participa, Wl_similar, bl_similar, Wr_similar):
    raise NotImplementedError("write your pallas kernel here")


# speedup vs baseline: 5.8006x; 5.8006x over previous
import jax
import jax.numpy as jnp
from jax import lax
from jax.experimental import pallas as pl
from jax.experimental.pallas import tpu as pltpu
from jax.experimental.pallas import tpu_sc as plsc

N = 10000
D = 128
H = 128
CH = 128
GB = 8
NC = 2
NS = 16
RQ = 80
NQ = N // RQ
BN = 1000


def _transform_body(x_ref, w_ref, y_ref):
    y_ref[0] = jnp.dot(x_ref[0], w_ref[0], preferred_element_type=jnp.float32)


def _combine_body(sum_ref, cnt_ref, x_ref, wp_ref, ws_ref, b_ref, o_ref):
    w = wp_ref[...] + ws_ref[...]
    z = jnp.dot(x_ref[...], w, preferred_element_type=jnp.float32)
    c0 = jnp.maximum(cnt_ref[0, :, 0:1], 1.0)
    c1 = jnp.maximum(cnt_ref[1, :, 0:1], 1.0)
    o_ref[...] = sum_ref[0] / c0 + sum_ref[1] / c1 + z + b_ref[...]


def _make_sc_body(nchunk):
    ngrp = nchunk // GB
    ntail = nchunk - ngrp * GB

    def _zero_fill(rows_v):
        @pl.loop(0, RQ)
        def _(i):
            @pl.loop(0, H // 16)
            def _(j):
                rows_v[i, pl.ds(j * 16, 16)] = jnp.zeros((16,), jnp.float32)

    def _zero_acc(s, rows_v, acc_sh):
        @pl.loop(0, (NQ + NS - 1) // NS)
        def _(j):
            q = j * NS + s

            @pl.when(q < NQ)
            def _():
                pltpu.sync_copy(rows_v.at[pl.ds(0, RQ)],
                                acc_sh.at[pl.ds(q * RQ, RQ)])

    def _flush(c, s, rows_v, acc_sh, out_hbm):
        @pl.loop(0, (NQ + NS - 1) // NS)
        def _(j):
            q = j * NS + s

            @pl.when(q < NQ)
            def _():
                b = q * RQ
                pltpu.sync_copy(acc_sh.at[pl.ds(b, RQ)], rows_v.at[pl.ds(0, RQ)])
                pltpu.sync_copy(rows_v.at[pl.ds(0, RQ)], out_hbm.at[c, pl.ds(b, RQ)])

    def body(src_hbm, dst_hbm, y_hbm, sum_hbm, cnt_hbm,
             acc_sh, sgrp_v, dgrp_v, rows_v, ones_v):
        c = lax.axis_index("c")
        s = lax.axis_index("s")

        _zero_fill(rows_v)

        @pl.loop(0, CH)
        def _(i):
            @pl.loop(0, H // 16)
            def _(j):
                ones_v[i, pl.ds(j * 16, 16)] = jnp.ones((16,), jnp.float32)

        _zero_acc(s, rows_v, acc_sh)
        plsc.subcore_barrier()

        @pl.loop(0, (ngrp + NS - 1) // NS)
        def _(j):
            g = j * NS + s

            @pl.when(g < ngrp)
            def _():
                pltpu.sync_copy(src_hbm.at[c, pl.ds(g * GB, GB)], sgrp_v)
                pltpu.sync_copy(dst_hbm.at[c, pl.ds(g * GB, GB)], dgrp_v)

                @pl.loop(0, GB)
                def _(r):
                    pltpu.sync_copy(y_hbm.at[sgrp_v.at[r]], rows_v)
                    pltpu.sync_copy(rows_v, acc_sh.at[dgrp_v.at[r]], add=True)

        if ntail:
            @pl.when(s == 0)
            def _():
                pltpu.sync_copy(src_hbm.at[c, pl.ds(ngrp * GB, ntail)],
                                sgrp_v.at[pl.ds(0, ntail)])
                pltpu.sync_copy(dst_hbm.at[c, pl.ds(ngrp * GB, ntail)],
                                dgrp_v.at[pl.ds(0, ntail)])
                for r in range(ntail):
                    pltpu.sync_copy(y_hbm.at[sgrp_v.at[r]], rows_v)
                    pltpu.sync_copy(rows_v, acc_sh.at[dgrp_v.at[r]], add=True)

        plsc.subcore_barrier()
        _flush(c, s, rows_v, acc_sh, sum_hbm)
        plsc.subcore_barrier()

        _zero_fill(rows_v)
        _zero_acc(s, rows_v, acc_sh)
        plsc.subcore_barrier()

        @pl.loop(0, (ngrp + NS - 1) // NS)
        def _(j):
            g = j * NS + s

            @pl.when(g < ngrp)
            def _():
                pltpu.sync_copy(dst_hbm.at[c, pl.ds(g * GB, GB)], dgrp_v)

                @pl.loop(0, GB)
                def _(r):
                    pltpu.sync_copy(ones_v, acc_sh.at[dgrp_v.at[r]], add=True)

        if ntail:
            @pl.when(s == 0)
            def _():
                pltpu.sync_copy(dst_hbm.at[c, pl.ds(ngrp * GB, ntail)],
                                dgrp_v.at[pl.ds(0, ntail)])
                for r in range(ntail):
                    pltpu.sync_copy(ones_v, acc_sh.at[dgrp_v.at[r]], add=True)

        plsc.subcore_barrier()
        _flush(c, s, rows_v, acc_sh, cnt_hbm)

    return body


def kernel(x_licitacion, x_empresa, ei_participa, ei_similar,
           Wl_participa, bl_participa, Wr_participa,
           Wl_similar, bl_similar, Wr_similar):
    E = ei_participa.shape[1]
    nchunk = E // CH

    xcat = jnp.stack([x_empresa, x_licitacion])
    wcat = jnp.stack([Wl_participa, Wl_similar])
    ycat = pl.pallas_call(
        _transform_body,
        grid=(2, N // BN),
        in_specs=[
            pl.BlockSpec((1, BN, D), lambda t, i: (t, i, 0)),
            pl.BlockSpec((1, D, H), lambda t, i: (t, 0, 0)),
        ],
        out_specs=pl.BlockSpec((1, BN, H), lambda t, i: (t, i, 0)),
        out_shape=jax.ShapeDtypeStruct((2, N, H), jnp.float32),
    )(xcat, wcat)
    ycat = ycat.reshape(2 * N, H)

    srccat = jnp.stack([ei_participa[0], ei_similar[0] + N]).reshape(2, nchunk, CH)
    dstcat = jnp.stack([ei_participa[1], ei_similar[1]]).reshape(2, nchunk, CH)

    mesh = plsc.VectorSubcoreMesh(core_axis_name="c", subcore_axis_name="s",
                                  num_cores=NC, num_subcores=NS)
    sums, cnts = pl.kernel(
        _make_sc_body(nchunk),
        out_type=(jax.ShapeDtypeStruct((2, N, H), jnp.float32),
                  jax.ShapeDtypeStruct((2, N, H), jnp.float32)),
        mesh=mesh,
        scratch_types=[
            pltpu.VMEM_SHARED((N, H), jnp.float32),
            pltpu.VMEM((GB, CH), jnp.int32),
            pltpu.VMEM((GB, CH), jnp.int32),
            pltpu.VMEM((CH, H), jnp.float32),
            pltpu.VMEM((CH, H), jnp.float32),
        ],
    )(srccat, dstcat, ycat)

    bsum = (bl_participa + bl_similar).reshape(1, H)
    out = pl.pallas_call(
        _combine_body,
        grid=(N // BN,),
        in_specs=[
            pl.BlockSpec((2, BN, H), lambda i: (0, i, 0)),
            pl.BlockSpec((2, BN, H), lambda i: (0, i, 0)),
            pl.BlockSpec((BN, D), lambda i: (i, 0)),
            pl.BlockSpec((D, H), lambda i: (0, 0)),
            pl.BlockSpec((D, H), lambda i: (0, 0)),
            pl.BlockSpec((1, H), lambda i: (0, 0)),
        ],
        out_specs=pl.BlockSpec((BN, H), lambda i: (i, 0)),
        out_shape=jax.ShapeDtypeStruct((N, H), jnp.float32),
    )(sums, cnts, x_licitacion, Wr_participa, Wr_similar, bsum)
    return out


# speedup vs baseline: 6.6626x; 1.1486x over previous
import jax
import jax.numpy as jnp
from jax import lax
from jax.experimental import pallas as pl
from jax.experimental.pallas import tpu as pltpu
from jax.experimental.pallas import tpu_sc as plsc

N = 10000
D = 128
H = 128
CH = 128
GB = 8
NC = 2
NS = 16
RQ = 80
NQ = N // RQ
BN = 1000


def _transform_body(x_ref, w_ref, y_ref):
    y_ref[0] = jnp.dot(x_ref[0], w_ref[0], preferred_element_type=jnp.float32)


def _combine_body(sum_ref, cnt_ref, x_ref, wp_ref, ws_ref, b_ref, o_ref):
    w = wp_ref[...] + ws_ref[...]
    z = jnp.dot(x_ref[...], w, preferred_element_type=jnp.float32)
    c0 = jnp.maximum(cnt_ref[0, :, 0:1], 1.0)
    c1 = jnp.maximum(cnt_ref[1, :, 0:1], 1.0)
    o_ref[...] = sum_ref[0] / c0 + sum_ref[1] / c1 + z + b_ref[...]


def _make_sc_body(nchunk):
    ngrp = nchunk // GB
    ntail = nchunk - ngrp * GB

    def _zero_fill(rows_v):
        @pl.loop(0, RQ)
        def _(i):
            @pl.loop(0, H // 16)
            def _(j):
                rows_v[i, pl.ds(j * 16, 16)] = jnp.zeros((16,), jnp.float32)

    def _zero_acc(s, rows_v, acc_sh):
        @pl.loop(0, (NQ + NS - 1) // NS)
        def _(j):
            q = j * NS + s

            @pl.when(q < NQ)
            def _():
                pltpu.sync_copy(rows_v.at[pl.ds(0, RQ)],
                                acc_sh.at[pl.ds(q * RQ, RQ)])

    def _flush(c, s, rows_v, acc_sh, out_hbm):
        @pl.loop(0, (NQ + NS - 1) // NS)
        def _(j):
            q = j * NS + s

            @pl.when(q < NQ)
            def _():
                b = q * RQ
                pltpu.sync_copy(acc_sh.at[pl.ds(b, RQ)], rows_v.at[pl.ds(0, RQ)])
                pltpu.sync_copy(rows_v.at[pl.ds(0, RQ)], out_hbm.at[c, pl.ds(b, RQ)])

    def body(src_hbm, dst_hbm, y_hbm, sum_hbm, cnt_hbm,
             acc_sh, sgrp_v, dgrp_v, rows_a, rows_b, sem_a, sem_b, sem_s):
        c = lax.axis_index("c")
        s = lax.axis_index("s")

        _zero_fill(rows_a)
        _zero_acc(s, rows_a, acc_sh)
        plsc.subcore_barrier()

        bufs = (rows_a, rows_b)
        sems = (sem_a, sem_b)

        @pl.loop(0, (ngrp + NS - 1) // NS)
        def _(j):
            g = j * NS + s

            @pl.when(g < ngrp)
            def _():
                pltpu.sync_copy(src_hbm.at[c, pl.ds(g * GB, GB)], sgrp_v)
                pltpu.sync_copy(dst_hbm.at[c, pl.ds(g * GB, GB)], dgrp_v)
                descs = {0: pltpu.async_copy(y_hbm.at[sgrp_v.at[0]],
                                             bufs[0], sems[0])}
                for r in range(GB):
                    descs[r].wait()
                    if r + 1 < GB:
                        descs[r + 1] = pltpu.async_copy(
                            y_hbm.at[sgrp_v.at[r + 1]],
                            bufs[(r + 1) % 2], sems[(r + 1) % 2])
                    pltpu.sync_copy(bufs[r % 2], acc_sh.at[dgrp_v.at[r]],
                                    add=True)

        if ntail:
            @pl.when(s == 0)
            def _():
                pltpu.sync_copy(src_hbm.at[c, pl.ds(ngrp * GB, ntail)],
                                sgrp_v.at[pl.ds(0, ntail)])
                pltpu.sync_copy(dst_hbm.at[c, pl.ds(ngrp * GB, ntail)],
                                dgrp_v.at[pl.ds(0, ntail)])
                for r in range(ntail):
                    pltpu.sync_copy(y_hbm.at[sgrp_v.at[r]], rows_a)
                    pltpu.sync_copy(rows_a, acc_sh.at[dgrp_v.at[r]], add=True)

        plsc.subcore_barrier()
        _flush(c, s, rows_a, acc_sh, sum_hbm)
        plsc.subcore_barrier()

        _zero_fill(rows_a)
        _zero_acc(s, rows_a, acc_sh)
        plsc.subcore_barrier()

        @pl.loop(0, CH)
        def _(i):
            @pl.loop(0, H // 16)
            def _(j):
                rows_a[i, pl.ds(j * 16, 16)] = jnp.ones((16,), jnp.float32)

        @pl.loop(0, (ngrp + NS - 1) // NS)
        def _(j):
            g = j * NS + s

            @pl.when(g < ngrp)
            def _():
                pltpu.sync_copy(dst_hbm.at[c, pl.ds(g * GB, GB)], dgrp_v)
                descs = [pltpu.async_copy(rows_a, acc_sh.at[dgrp_v.at[r]],
                                          sem_s, add=True)
                         for r in range(GB)]
                for d in descs:
                    d.wait()

        if ntail:
            @pl.when(s == 0)
            def _():
                pltpu.sync_copy(dst_hbm.at[c, pl.ds(ngrp * GB, ntail)],
                                dgrp_v.at[pl.ds(0, ntail)])
                for r in range(ntail):
                    pltpu.sync_copy(rows_a, acc_sh.at[dgrp_v.at[r]], add=True)

        plsc.subcore_barrier()
        _flush(c, s, rows_b, acc_sh, cnt_hbm)

    return body


def kernel(x_licitacion, x_empresa, ei_participa, ei_similar,
           Wl_participa, bl_participa, Wr_participa,
           Wl_similar, bl_similar, Wr_similar):
    E = ei_participa.shape[1]
    nchunk = E // CH

    xcat = jnp.stack([x_empresa, x_licitacion])
    wcat = jnp.stack([Wl_participa, Wl_similar])
    ycat = pl.pallas_call(
        _transform_body,
        grid=(2, N // BN),
        in_specs=[
            pl.BlockSpec((1, BN, D), lambda t, i: (t, i, 0)),
            pl.BlockSpec((1, D, H), lambda t, i: (t, 0, 0)),
        ],
        out_specs=pl.BlockSpec((1, BN, H), lambda t, i: (t, i, 0)),
        out_shape=jax.ShapeDtypeStruct((2, N, H), jnp.float32),
    )(xcat, wcat)
    ycat = ycat.reshape(2 * N, H)

    srccat = jnp.stack([ei_participa[0], ei_similar[0] + N]).reshape(2, nchunk, CH)
    dstcat = jnp.stack([ei_participa[1], ei_similar[1]]).reshape(2, nchunk, CH)

    mesh = plsc.VectorSubcoreMesh(core_axis_name="c", subcore_axis_name="s",
                                  num_cores=NC, num_subcores=NS)
    sums, cnts = pl.kernel(
        _make_sc_body(nchunk),
        out_type=(jax.ShapeDtypeStruct((2, N, H), jnp.float32),
                  jax.ShapeDtypeStruct((2, N, H), jnp.float32)),
        mesh=mesh,
        scratch_types=[
            pltpu.VMEM_SHARED((N, H), jnp.float32),
            pltpu.VMEM((GB, CH), jnp.int32),
            pltpu.VMEM((GB, CH), jnp.int32),
            pltpu.VMEM((CH, H), jnp.float32),
            pltpu.VMEM((CH, H), jnp.float32),
            pltpu.SemaphoreType.DMA,
            pltpu.SemaphoreType.DMA,
            pltpu.SemaphoreType.DMA,
        ],
    )(srccat, dstcat, ycat)

    bsum = (bl_participa + bl_similar).reshape(1, H)
    out = pl.pallas_call(
        _combine_body,
        grid=(N // BN,),
        in_specs=[
            pl.BlockSpec((2, BN, H), lambda i: (0, i, 0)),
            pl.BlockSpec((2, BN, H), lambda i: (0, i, 0)),
            pl.BlockSpec((BN, D), lambda i: (i, 0)),
            pl.BlockSpec((D, H), lambda i: (0, 0)),
            pl.BlockSpec((D, H), lambda i: (0, 0)),
            pl.BlockSpec((1, H), lambda i: (0, 0)),
        ],
        out_specs=pl.BlockSpec((BN, H), lambda i: (i, 0)),
        out_shape=jax.ShapeDtypeStruct((N, H), jnp.float32),
    )(sums, cnts, x_licitacion, Wr_participa, Wr_similar, bsum)
    return out


# speedup vs baseline: 6.7180x; 1.0083x over previous
import jax
import jax.numpy as jnp
from jax import lax
from jax.experimental import pallas as pl
from jax.experimental.pallas import tpu as pltpu
from jax.experimental.pallas import tpu_sc as plsc

N = 10000
D = 128
H = 128
CH = 128
GB = 8
NC = 2
NS = 16
RQ = 80
NQ = N // RQ
RQF = 200
NQF = N // RQF
BN = 1000


def _transform_body(x_ref, w_ref, y_ref):
    y_ref[0] = jnp.dot(x_ref[0], w_ref[0], preferred_element_type=jnp.float32)


def _combine_body(sum_ref, cnt_ref, x_ref, wp_ref, ws_ref, b_ref, o_ref):
    w = wp_ref[...] + ws_ref[...]
    z = jnp.dot(x_ref[...], w, preferred_element_type=jnp.float32)
    c0 = jnp.maximum(cnt_ref[0, :, 0:1], 1.0)
    c1 = jnp.maximum(cnt_ref[1, :, 0:1], 1.0)
    o_ref[...] = sum_ref[0] / c0 + sum_ref[1] / c1 + z + b_ref[...]


def _make_sc_body(nchunk):
    ngrp = nchunk // GB
    ntail = nchunk - ngrp * GB

    def _zero_fill(rows_v):
        @pl.loop(0, RQ)
        def _(i):
            @pl.loop(0, H // 16)
            def _(j):
                rows_v[i, pl.ds(j * 16, 16)] = jnp.zeros((16,), jnp.float32)

    def _zero_acc(s, rows_v, acc_sh):
        @pl.loop(0, (NQ + NS - 1) // NS)
        def _(j):
            q = j * NS + s

            @pl.when(q < NQ)
            def _():
                pltpu.sync_copy(rows_v.at[pl.ds(0, RQ)],
                                acc_sh.at[pl.ds(q * RQ, RQ)])

    def _flush(c, s, acc_sh, out_hbm):
        @pl.loop(0, (NQF + NS - 1) // NS)
        def _(j):
            q = j * NS + s

            @pl.when(q < NQF)
            def _():
                b = q * RQF
                pltpu.sync_copy(acc_sh.at[pl.ds(b, RQF)], out_hbm.at[c, pl.ds(b, RQF)])

    def body(src_hbm, dst_hbm, y_hbm, sum_hbm, cnt_hbm,
             acc_sh, sgrp_v, dgrp_v, rows_a, rows_b, sem_a, sem_b, sem_s):
        c = lax.axis_index("c")
        s = lax.axis_index("s")

        _zero_fill(rows_a)
        _zero_acc(s, rows_a, acc_sh)
        plsc.subcore_barrier()

        bufs = (rows_a, rows_b)
        sems = (sem_a, sem_b)

        @pl.loop(0, (ngrp + NS - 1) // NS)
        def _(j):
            g = j * NS + s

            @pl.when(g < ngrp)
            def _():
                pltpu.sync_copy(src_hbm.at[c, pl.ds(g * GB, GB)], sgrp_v)
                pltpu.sync_copy(dst_hbm.at[c, pl.ds(g * GB, GB)], dgrp_v)
                gd = {0: pltpu.async_copy(y_hbm.at[sgrp_v.at[0]],
                                          bufs[0], sems[0])}
                sd = {}
                for r in range(GB):
                    gd[r].wait()
                    if r + 1 < GB:
                        if r >= 1:
                            sd[r - 1].wait()
                        gd[r + 1] = pltpu.async_copy(
                            y_hbm.at[sgrp_v.at[r + 1]],
                            bufs[(r + 1) % 2], sems[(r + 1) % 2])
                    sd[r] = pltpu.async_copy(bufs[r % 2],
                                             acc_sh.at[dgrp_v.at[r]],
                                             sem_s, add=True)
                sd[GB - 2].wait()
                sd[GB - 1].wait()

        if ntail:
            @pl.when(s == 0)
            def _():
                pltpu.sync_copy(src_hbm.at[c, pl.ds(ngrp * GB, ntail)],
                                sgrp_v.at[pl.ds(0, ntail)])
                pltpu.sync_copy(dst_hbm.at[c, pl.ds(ngrp * GB, ntail)],
                                dgrp_v.at[pl.ds(0, ntail)])
                for r in range(ntail):
                    pltpu.sync_copy(y_hbm.at[sgrp_v.at[r]], rows_a)
                    pltpu.sync_copy(rows_a, acc_sh.at[dgrp_v.at[r]], add=True)

        plsc.subcore_barrier()
        _flush(c, s, acc_sh, sum_hbm)
        plsc.subcore_barrier()

        _zero_fill(rows_a)
        _zero_acc(s, rows_a, acc_sh)
        plsc.subcore_barrier()

        @pl.loop(0, CH)
        def _(i):
            @pl.loop(0, H // 16)
            def _(j):
                rows_a[i, pl.ds(j * 16, 16)] = jnp.ones((16,), jnp.float32)

        @pl.loop(0, (ngrp + NS - 1) // NS)
        def _(j):
            g = j * NS + s

            @pl.when(g < ngrp)
            def _():
                pltpu.sync_copy(dst_hbm.at[c, pl.ds(g * GB, GB)], dgrp_v)
                descs = [pltpu.async_copy(rows_a, acc_sh.at[dgrp_v.at[r]],
                                          sem_s, add=True)
                         for r in range(GB)]
                for d in descs:
                    d.wait()

        if ntail:
            @pl.when(s == 0)
            def _():
                pltpu.sync_copy(dst_hbm.at[c, pl.ds(ngrp * GB, ntail)],
                                dgrp_v.at[pl.ds(0, ntail)])
                for r in range(ntail):
                    pltpu.sync_copy(rows_a, acc_sh.at[dgrp_v.at[r]], add=True)

        plsc.subcore_barrier()
        _flush(c, s, acc_sh, cnt_hbm)

    return body


def kernel(x_licitacion, x_empresa, ei_participa, ei_similar,
           Wl_participa, bl_participa, Wr_participa,
           Wl_similar, bl_similar, Wr_similar):
    E = ei_participa.shape[1]
    nchunk = E // CH

    xcat = jnp.stack([x_empresa, x_licitacion])
    wcat = jnp.stack([Wl_participa, Wl_similar])
    ycat = pl.pallas_call(
        _transform_body,
        grid=(2, N // BN),
        in_specs=[
            pl.BlockSpec((1, BN, D), lambda t, i: (t, i, 0)),
            pl.BlockSpec((1, D, H), lambda t, i: (t, 0, 0)),
        ],
        out_specs=pl.BlockSpec((1, BN, H), lambda t, i: (t, i, 0)),
        out_shape=jax.ShapeDtypeStruct((2, N, H), jnp.float32),
    )(xcat, wcat)
    ycat = ycat.reshape(2 * N, H)

    srccat = jnp.stack([ei_participa[0], ei_similar[0] + N]).reshape(2, nchunk, CH)
    dstcat = jnp.stack([ei_participa[1], ei_similar[1]]).reshape(2, nchunk, CH)

    mesh = plsc.VectorSubcoreMesh(core_axis_name="c", subcore_axis_name="s",
                                  num_cores=NC, num_subcores=NS)
    sums, cnts = pl.kernel(
        _make_sc_body(nchunk),
        out_type=(jax.ShapeDtypeStruct((2, N, H), jnp.float32),
                  jax.ShapeDtypeStruct((2, N, H), jnp.float32)),
        mesh=mesh,
        scratch_types=[
            pltpu.VMEM_SHARED((N, H), jnp.float32),
            pltpu.VMEM((GB, CH), jnp.int32),
            pltpu.VMEM((GB, CH), jnp.int32),
            pltpu.VMEM((CH, H), jnp.float32),
            pltpu.VMEM((CH, H), jnp.float32),
            pltpu.SemaphoreType.DMA,
            pltpu.SemaphoreType.DMA,
            pltpu.SemaphoreType.DMA,
        ],
    )(srccat, dstcat, ycat)

    bsum = (bl_participa + bl_similar).reshape(1, H)
    out = pl.pallas_call(
        _combine_body,
        grid=(N // BN,),
        in_specs=[
            pl.BlockSpec((2, BN, H), lambda i: (0, i, 0)),
            pl.BlockSpec((2, BN, H), lambda i: (0, i, 0)),
            pl.BlockSpec((BN, D), lambda i: (i, 0)),
            pl.BlockSpec((D, H), lambda i: (0, 0)),
            pl.BlockSpec((D, H), lambda i: (0, 0)),
            pl.BlockSpec((1, H), lambda i: (0, 0)),
        ],
        out_specs=pl.BlockSpec((BN, H), lambda i: (i, 0)),
        out_shape=jax.ShapeDtypeStruct((N, H), jnp.float32),
    )(sums, cnts, x_licitacion, Wr_participa, Wr_similar, bsum)
    return out
